# baseline (speedup 1.0000x reference)
import jax
import jax.numpy as jnp
from jax.experimental import pallas as pl
from jax.experimental.pallas import tpu as pltpu


_LANES = 128
_ROWS_PER_CH = (64 * 64) // _LANES
_CHUNK_ROWS = 2048
_DEPTH = 6


def _exchange_body(a_hbm, b_hbm, o1_hbm, o2_hbm,
                   buf_a, buf_b, buf_o1, buf_o2,
                   sem_in_a, sem_in_b, sem_o1, sem_o2):
    nitems = a_hbm.shape[0] // _CHUNK_ROWS

    def in_copies(i, slot):
        sl = pl.ds(i * _CHUNK_ROWS, _CHUNK_ROWS)
        return (
            pltpu.make_async_copy(a_hbm.at[sl], buf_a.at[slot], sem_in_a.at[slot]),
            pltpu.make_async_copy(b_hbm.at[sl], buf_b.at[slot], sem_in_b.at[slot]),
        )

    def out_copies(i, slot):
        sl = pl.ds(i * _CHUNK_ROWS, _CHUNK_ROWS)
        return (
            pltpu.make_async_copy(buf_o1.at[slot], o1_hbm.at[sl], sem_o1.at[slot]),
            pltpu.make_async_copy(buf_o2.at[slot], o2_hbm.at[sl], sem_o2.at[slot]),
        )

    row = jax.lax.broadcasted_iota(jnp.int32, (_CHUNK_ROWS, _LANES), 0)
    mask = ((row // _ROWS_PER_CH) % 2) == 0

    for i in range(_DEPTH):
        for cp in in_copies(i, i % _DEPTH):
            cp.start()

    for i in range(nitems):
        slot = i % _DEPTH
        if i >= _DEPTH:
            for cp in out_copies(i - _DEPTH, slot):
                cp.wait()
        for cp in in_copies(i, slot):
            cp.wait()
        a = buf_a[slot]
        b = buf_b[slot]
        buf_o1[slot] = jnp.where(mask, b, a)
        buf_o2[slot] = jnp.where(mask, a, b)
        for cp in out_copies(i, slot):
            cp.start()
        if i + _DEPTH < nitems:
            for cp in in_copies(i + _DEPTH, slot):
                cp.start()

    for i in range(nitems - _DEPTH, nitems):
        for cp in out_copies(i, i % _DEPTH):
            cp.wait()


def kernel(x1, x2):
    N, c, h, w = x1.shape
    rows = (N * c * h * w) // _LANES
    a = x1.reshape(rows, _LANES)
    b = x2.reshape(rows, _LANES)
    spec = pl.BlockSpec(memory_space=pl.ANY)
    o1, o2 = pl.pallas_call(
        _exchange_body,
        in_specs=[spec, spec],
        out_specs=[spec, spec],
        out_shape=[
            jax.ShapeDtypeStruct((rows, _LANES), x1.dtype),
            jax.ShapeDtypeStruct((rows, _LANES), x2.dtype),
        ],
        scratch_shapes=[
            pltpu.VMEM((_DEPTH, _CHUNK_ROWS, _LANES), x1.dtype),
            pltpu.VMEM((_DEPTH, _CHUNK_ROWS, _LANES), x1.dtype),
            pltpu.VMEM((_DEPTH, _CHUNK_ROWS, _LANES), x1.dtype),
            pltpu.VMEM((_DEPTH, _CHUNK_ROWS, _LANES), x1.dtype),
            pltpu.SemaphoreType.DMA((_DEPTH,)),
            pltpu.SemaphoreType.DMA((_DEPTH,)),
            pltpu.SemaphoreType.DMA((_DEPTH,)),
            pltpu.SemaphoreType.DMA((_DEPTH,)),
        ],
    )(a, b)
    return (o1.reshape(N, c, h, w), o2.reshape(N, c, h, w))
